# baseline (speedup 1.0000x reference)
import functools

import jax
import jax.numpy as jnp
from jax import lax
from jax.experimental import pallas as pl
from jax.experimental.layout import Layout, with_layout_constraint
from jax.experimental.pallas import tpu as pltpu
from jax.experimental.pallas import tpu_sc as plsc

_MU = 3.5
_B = 16384
_D = 64
_NU = 1000000
_NI = 100000
_NC = 2
_NS = 16
_NW = _NC * _NS
_BPW = _B // _NW
_NP = 2
_BPC = _BPW // _NP
_L = 16
_NG = _BPC // _L


def _win(idx, n):
    return jnp.minimum(idx & -8, n - _L)


def _sc_body(eu_hbm, ei_hbm, ub_hbm, ib_hbm, uidx_hbm, iidx_hbm, out_hbm,
             uidx_v, iidx_v, eu_c, ei_c, ub_w, ib_w, out_v,
             sem, bsem):
    wid = lax.axis_index("s") * _NC + lax.axis_index("c")
    base = wid * _BPW

    pltpu.sync_copy(uidx_hbm.at[pl.ds(base, _BPW)], uidx_v)
    pltpu.sync_copy(iidx_hbm.at[pl.ds(base, _BPW)], iidx_v)

    lane = lax.iota(jnp.int32, _L)

    for p in range(_NP):
        p0 = p * _BPC

        def fire(g, carry):
            uv = uidx_v[pl.ds(p0 + g * _L, _L)]
            iv = iidx_v[pl.ds(p0 + g * _L, _L)]
            uwin = _win(uv, _NU)
            iwin = _win(iv, _NI)
            for l in range(_L):
                r = g * _L + l
                u = uv[l]
                i = iv[l]
                r16 = pl.multiple_of((p0 + r) * _L, 8)
                pltpu.async_copy(eu_hbm.at[pl.ds(u, 1), :],
                                 eu_c.at[pl.ds(r, 1), :], sem)
                pltpu.async_copy(ei_hbm.at[pl.ds(i, 1), :],
                                 ei_c.at[pl.ds(r, 1), :], sem)
                pltpu.async_copy(
                    ub_hbm.at[pl.ds(pl.multiple_of(uwin[l], 8), _L)],
                    ub_w.at[pl.ds(r16, _L)], bsem)
                pltpu.async_copy(
                    ib_hbm.at[pl.ds(pl.multiple_of(iwin[l], 8), _L)],
                    ib_w.at[pl.ds(r16, _L)], bsem)
            return carry

        lax.fori_loop(0, _NG, fire, 0)

        pltpu.make_async_copy(eu_hbm.at[pl.ds(0, _BPC), :], eu_c, sem).wait()
        pltpu.make_async_copy(ei_hbm.at[pl.ds(0, _BPC), :], ei_c, sem).wait()
        pltpu.make_async_copy(ub_hbm.at[pl.ds(0, _BPC * _L)],
                              ub_w.at[pl.ds(p0 * _L, _BPC * _L)], bsem).wait()
        pltpu.make_async_copy(ib_hbm.at[pl.ds(0, _BPC * _L)],
                              ib_w.at[pl.ds(p0 * _L, _BPC * _L)], bsem).wait()

        def group(g, carry):
            w = jnp.zeros((_L,), jnp.float32)
            for l in range(_L):
                r = g * _L + l
                acc = eu_c[r, pl.ds(0, _L)] * ei_c[r, pl.ds(0, _L)]
                for c in range(1, _D // _L):
                    acc = acc + (eu_c[r, pl.ds(c * _L, _L)]
                                 * ei_c[r, pl.ds(c * _L, _L)])
                s = jnp.sum(acc)
                w = jnp.where(lane == l, jnp.full((_L,), s), w)
            uvec = uidx_v[pl.ds(p0 + g * _L, _L)]
            ivec = iidx_v[pl.ds(p0 + g * _L, _L)]
            base16 = (lane + p0 + g * _L) * _L
            ub_vals = plsc.load_gather(ub_w, [base16 + (uvec - _win(uvec, _NU))])
            ib_vals = plsc.load_gather(ib_w, [base16 + (ivec - _win(ivec, _NI))])
            out_v[pl.ds(p0 + g * _L, _L)] = w + ub_vals + ib_vals + _MU
            return carry

        lax.fori_loop(0, _NG, group, 0)

    pltpu.sync_copy(out_v, out_hbm.at[pl.ds(base, _BPW)])


@jax.jit
def _sc_call(embed_user, embed_item, ub_flat, ib_flat, user_idx, item_idx):
    mesh = plsc.VectorSubcoreMesh(core_axis_name="c", subcore_axis_name="s")
    run = functools.partial(
        pl.kernel,
        mesh=mesh,
        compiler_params=pltpu.CompilerParams(
            needs_layout_passes=False, use_tc_tiling_on_sc=True),
        out_type=jax.ShapeDtypeStruct((_B,), jnp.float32),
        scratch_types=[
            pltpu.VMEM((_BPW,), jnp.int32),
            pltpu.VMEM((_BPW,), jnp.int32),
            pltpu.VMEM((_BPC, _D), jnp.float32),
            pltpu.VMEM((_BPC, _D), jnp.float32),
            pltpu.VMEM((_BPW * _L,), jnp.float32),
            pltpu.VMEM((_BPW * _L,), jnp.float32),
            pltpu.VMEM((_BPW,), jnp.float32),
            pltpu.SemaphoreType.DMA,
            pltpu.SemaphoreType.DMA,
        ],
    )(_sc_body)
    return run(embed_user, embed_item, ub_flat, ib_flat, user_idx, item_idx)


def kernel(embed_user, embed_item, user_bias, item_bias, user_idx, item_idx):
    native = Layout(major_to_minor=(1, 0), tiling=((8, 128),))
    embed_user = with_layout_constraint(embed_user, native)
    embed_item = with_layout_constraint(embed_item, native)
    return _sc_call(embed_user, embed_item,
                    user_bias.reshape(-1), item_bias.reshape(-1),
                    user_idx.astype(jnp.int32), item_idx.astype(jnp.int32))


# speedup vs baseline: 1.0051x; 1.0051x over previous
import functools

import jax
import jax.numpy as jnp
from jax import lax
from jax.experimental import pallas as pl
from jax.experimental.layout import Layout, with_layout_constraint
from jax.experimental.pallas import tpu as pltpu
from jax.experimental.pallas import tpu_sc as plsc

_MU = 3.5
_B = 16384
_D = 64
_NU = 1000000
_NI = 100000
_NC = 2
_NS = 16
_NW = _NC * _NS
_BPW = _B // _NW
_NP = 2
_BPC = _BPW // _NP
_L = 16
_NG = _BPC // _L


def _win(idx, n):
    return jnp.minimum(idx & -8, n - _L)


def _sc_body(eu_hbm, ei_hbm, ub_hbm, ib_hbm, uidx_hbm, iidx_hbm, out_hbm,
             uidx_v, iidx_v, eu_c, ei_c, ub_w, ib_w, out_v,
             sem, bsem):
    wid = lax.axis_index("s") * _NC + lax.axis_index("c")
    base = wid * _BPW

    pltpu.sync_copy(uidx_hbm.at[pl.ds(base, _BPW)], uidx_v)
    pltpu.sync_copy(iidx_hbm.at[pl.ds(base, _BPW)], iidx_v)

    lane = lax.iota(jnp.int32, _L)

    for p in range(_NP):
        p0 = p * _BPC

        def fire(g, carry):
            uv = uidx_v[pl.ds(p0 + g * _L, _L)]
            iv = iidx_v[pl.ds(p0 + g * _L, _L)]
            uwin = _win(uv, _NU)
            iwin = _win(iv, _NI)
            for l in range(_L):
                r = g * _L + l
                u = uv[l]
                i = iv[l]
                r16 = pl.multiple_of((p0 + r) * _L, 8)
                r64 = pl.multiple_of(r * _D, 8)
                pltpu.async_copy(eu_hbm.at[u],
                                 eu_c.at[pl.ds(r64, _D)], sem)
                pltpu.async_copy(ei_hbm.at[i],
                                 ei_c.at[pl.ds(r64, _D)], sem)
                pltpu.async_copy(
                    ub_hbm.at[pl.ds(pl.multiple_of(uwin[l], 8), _L)],
                    ub_w.at[pl.ds(r16, _L)], bsem)
                pltpu.async_copy(
                    ib_hbm.at[pl.ds(pl.multiple_of(iwin[l], 8), _L)],
                    ib_w.at[pl.ds(r16, _L)], bsem)
            return carry

        lax.fori_loop(0, _NG, fire, 0)

        pltpu.make_async_copy(ub_hbm.at[pl.ds(0, _BPC * _D)], eu_c, sem).wait()
        pltpu.make_async_copy(ub_hbm.at[pl.ds(0, _BPC * _D)], ei_c, sem).wait()
        pltpu.make_async_copy(ub_hbm.at[pl.ds(0, _BPC * _L)],
                              ub_w.at[pl.ds(p0 * _L, _BPC * _L)], bsem).wait()
        pltpu.make_async_copy(ib_hbm.at[pl.ds(0, _BPC * _L)],
                              ib_w.at[pl.ds(p0 * _L, _BPC * _L)], bsem).wait()

        def group(g, carry):
            w = jnp.zeros((_L,), jnp.float32)
            for l in range(_L):
                r = g * _L + l
                def ch(c):
                    return pl.ds(pl.multiple_of(r * _D + c * _L, 8), _L)
                acc = eu_c[ch(0)] * ei_c[ch(0)]
                for c in range(1, _D // _L):
                    acc = acc + eu_c[ch(c)] * ei_c[ch(c)]
                s = jnp.sum(acc)
                w = jnp.where(lane == l, jnp.full((_L,), s), w)
            uvec = uidx_v[pl.ds(p0 + g * _L, _L)]
            ivec = iidx_v[pl.ds(p0 + g * _L, _L)]
            base16 = (lane + p0 + g * _L) * _L
            ub_vals = plsc.load_gather(ub_w, [base16 + (uvec - _win(uvec, _NU))])
            ib_vals = plsc.load_gather(ib_w, [base16 + (ivec - _win(ivec, _NI))])
            out_v[pl.ds(p0 + g * _L, _L)] = w + ub_vals + ib_vals + _MU
            return carry

        lax.fori_loop(0, _NG, group, 0)

    pltpu.sync_copy(out_v, out_hbm.at[pl.ds(base, _BPW)])


@jax.jit
def _sc_call(embed_user, embed_item, ub_flat, ib_flat, user_idx, item_idx):
    mesh = plsc.VectorSubcoreMesh(core_axis_name="c", subcore_axis_name="s")
    run = functools.partial(
        pl.kernel,
        mesh=mesh,
        compiler_params=pltpu.CompilerParams(
            needs_layout_passes=False, use_tc_tiling_on_sc=True),
        out_type=jax.ShapeDtypeStruct((_B,), jnp.float32),
        scratch_types=[
            pltpu.VMEM((_BPW,), jnp.int32),
            pltpu.VMEM((_BPW,), jnp.int32),
            pltpu.VMEM((_BPC * _D,), jnp.float32),
            pltpu.VMEM((_BPC * _D,), jnp.float32),
            pltpu.VMEM((_BPW * _L,), jnp.float32),
            pltpu.VMEM((_BPW * _L,), jnp.float32),
            pltpu.VMEM((_BPW,), jnp.float32),
            pltpu.SemaphoreType.DMA,
            pltpu.SemaphoreType.DMA,
        ],
    )(_sc_body)
    return run(embed_user, embed_item, ub_flat, ib_flat, user_idx, item_idx)


def kernel(embed_user, embed_item, user_bias, item_bias, user_idx, item_idx):
    row_major = Layout(major_to_minor=(0, 1), tiling=((8, 128),))
    embed_user = with_layout_constraint(embed_user, row_major)
    embed_item = with_layout_constraint(embed_item, row_major)
    return _sc_call(embed_user, embed_item,
                    user_bias.reshape(-1), item_bias.reshape(-1),
                    user_idx.astype(jnp.int32), item_idx.astype(jnp.int32))


# speedup vs baseline: 1.9443x; 1.9345x over previous
import functools

import jax
import jax.numpy as jnp
from jax import lax
from jax.experimental import pallas as pl
from jax.experimental.pallas import tpu as pltpu
from jax.experimental.pallas import tpu_sc as plsc

_MU = 3.5
_B = 16384
_D = 64
_NU = 1000000
_NI = 100000
_NC = 2
_NS = 16
_NW = _NC * _NS
_BPW = _B // _NW
_NP = 2
_BPC = _BPW // _NP
_L = 16
_NG = _BPC // _L
_CPN = 10


def _win(idx, n):
    return jnp.minimum(idx & -8, n - _L)


def _extract(vec_ref, k):
    base = pl.multiple_of(jnp.minimum(k & -8, vec_ref.shape[0] - _L), 8)
    v = vec_ref[pl.ds(base, _L)]
    return v.at[jnp.full((_L,), k - base, jnp.int32)].get(
        mode="promise_in_bounds")[0]


def _make_streamer(n):
    p_phys = -(-n // 128)
    chunk_words = _CPN * 1024

    def body(tbl, sidx, out_hbm, uv, pbuf, rows_out, sem):
        lane = lax.iota(jnp.int32, _L)
        wid = lax.axis_index("s") * _NC + lax.axis_index("c")
        seg = wid * _BPW
        pltpu.sync_copy(sidx.at[pl.ds(seg, _BPW)], uv)

        def fetch(pb):
            base = pl.multiple_of(pb * 128, 128)
            for c8 in range(8):
                pltpu.async_copy(
                    tbl.at[pl.ds(c8 * 8, 8), pl.ds(base, _CPN * 128)],
                    pbuf.at[pl.ds(c8 * 8, 8), :], sem)
            pltpu.make_async_copy(
                tbl.at[pl.ds(0, _D), pl.ds(0, _CPN * 128)], pbuf, sem).wait()

        def inner_cond(c):
            k, u_k, pend = c
            return jnp.logical_and(k < _BPW, u_k < pend)

        def inner(c):
            k, u_k, pend = c
            u_rel = u_k - (pend - _CPN * 128)
            cols = jnp.full((_L,), u_rel, jnp.int32)
            for t in range(_D // _L):
                vals = plsc.load_gather(pbuf, [t * _L + lane, cols])
                rows_out[pl.ds(pl.multiple_of(k * _D + t * _L, 16), _L)] = vals
            k = k + 1
            u_k = _extract(uv, jnp.minimum(k, _BPW - 1))
            return k, u_k, pend

        def outer_cond(c):
            return c[0] < _BPW

        def outer(c):
            k, u_k = c
            pb = jnp.minimum(u_k >> 7, p_phys - _CPN)
            fetch(pb)
            k, u_k, _ = lax.while_loop(
                inner_cond, inner, (k, u_k, (pb + _CPN) * 128))
            return k, u_k

        k0 = jnp.int32(0)
        lax.while_loop(outer_cond, outer, (k0, _extract(uv, k0)))
        pltpu.sync_copy(rows_out, out_hbm.at[pl.ds(seg * _D, _BPW * _D)])

    mesh = plsc.VectorSubcoreMesh(core_axis_name="c", subcore_axis_name="s")
    return functools.partial(
        pl.kernel,
        mesh=mesh,
        compiler_params=pltpu.CompilerParams(
            needs_layout_passes=False, use_tc_tiling_on_sc=True),
        out_type=jax.ShapeDtypeStruct((_B * _D,), jnp.float32),
        scratch_types=[
            pltpu.VMEM((_BPW,), jnp.int32),
            pltpu.VMEM((_D, _CPN * 128), jnp.float32),
            pltpu.VMEM((_BPW * _D,), jnp.float32),
            pltpu.SemaphoreType.DMA,
        ],
    )(body)


def _dot_body(eu_hbm, ei_hbm, ub_hbm, ib_hbm, posu_hbm, posi_hbm,
              uidx_hbm, iidx_hbm, out_hbm,
              posu_v, posi_v, uidx_v, iidx_v, eu_c, ei_c, ub_w, ib_w, out_v,
              sem, bsem):
    wid = lax.axis_index("s") * _NC + lax.axis_index("c")
    base = wid * _BPW

    pltpu.sync_copy(posu_hbm.at[pl.ds(base, _BPW)], posu_v)
    pltpu.sync_copy(posi_hbm.at[pl.ds(base, _BPW)], posi_v)
    pltpu.sync_copy(uidx_hbm.at[pl.ds(base, _BPW)], uidx_v)
    pltpu.sync_copy(iidx_hbm.at[pl.ds(base, _BPW)], iidx_v)

    lane = lax.iota(jnp.int32, _L)

    for p in range(_NP):
        p0 = p * _BPC

        def fire(g, carry):
            pu = posu_v[pl.ds(p0 + g * _L, _L)]
            pi = posi_v[pl.ds(p0 + g * _L, _L)]
            uv = uidx_v[pl.ds(p0 + g * _L, _L)]
            iv = iidx_v[pl.ds(p0 + g * _L, _L)]
            uwin = _win(uv, _NU)
            iwin = _win(iv, _NI)
            for l in range(_L):
                r = g * _L + l
                r16 = pl.multiple_of((p0 + r) * _L, 8)
                r64 = pl.multiple_of(r * _D, 8)
                pltpu.async_copy(
                    eu_hbm.at[pl.ds(pl.multiple_of(pu[l] * _D, 8), _D)],
                    eu_c.at[pl.ds(r64, _D)], sem)
                pltpu.async_copy(
                    ei_hbm.at[pl.ds(pl.multiple_of(pi[l] * _D, 8), _D)],
                    ei_c.at[pl.ds(r64, _D)], sem)
                pltpu.async_copy(
                    ub_hbm.at[pl.ds(pl.multiple_of(uwin[l], 8), _L)],
                    ub_w.at[pl.ds(r16, _L)], bsem)
                pltpu.async_copy(
                    ib_hbm.at[pl.ds(pl.multiple_of(iwin[l], 8), _L)],
                    ib_w.at[pl.ds(r16, _L)], bsem)
            return carry

        lax.fori_loop(0, _NG, fire, 0)

        pltpu.make_async_copy(ub_hbm.at[pl.ds(0, _BPC * _D)], eu_c, sem).wait()
        pltpu.make_async_copy(ub_hbm.at[pl.ds(0, _BPC * _D)], ei_c, sem).wait()
        pltpu.make_async_copy(ub_hbm.at[pl.ds(0, _BPC * _L)],
                              ub_w.at[pl.ds(p0 * _L, _BPC * _L)], bsem).wait()
        pltpu.make_async_copy(ib_hbm.at[pl.ds(0, _BPC * _L)],
                              ib_w.at[pl.ds(p0 * _L, _BPC * _L)], bsem).wait()

        def group(g, carry):
            w = jnp.zeros((_L,), jnp.float32)
            for l in range(_L):
                r = g * _L + l
                def ch(c):
                    return pl.ds(pl.multiple_of(r * _D + c * _L, 8), _L)
                acc = eu_c[ch(0)] * ei_c[ch(0)]
                for c in range(1, _D // _L):
                    acc = acc + eu_c[ch(c)] * ei_c[ch(c)]
                s = jnp.sum(acc)
                w = jnp.where(lane == l, jnp.full((_L,), s), w)
            uvec = uidx_v[pl.ds(p0 + g * _L, _L)]
            ivec = iidx_v[pl.ds(p0 + g * _L, _L)]
            base16 = (lane + p0 + g * _L) * _L
            ub_vals = plsc.load_gather(ub_w, [base16 + (uvec - _win(uvec, _NU))])
            ib_vals = plsc.load_gather(ib_w, [base16 + (ivec - _win(ivec, _NI))])
            out_v[pl.ds(p0 + g * _L, _L)] = w + ub_vals + ib_vals + _MU
            return carry

        lax.fori_loop(0, _NG, group, 0)

    pltpu.sync_copy(out_v, out_hbm.at[pl.ds(base, _BPW)])


@jax.jit
def _sc_call(eu_t, ei_t, ub_flat, ib_flat, user_idx, item_idx):
    su_sorted = jnp.sort(user_idx)
    si_sorted = jnp.sort(item_idx)
    pu = jnp.argsort(user_idx)
    pi = jnp.argsort(item_idx)
    arange = jnp.arange(_B, dtype=jnp.int32)
    posu = jnp.zeros((_B,), jnp.int32).at[pu].set(arange)
    posi = jnp.zeros((_B,), jnp.int32).at[pi].set(arange)

    eu_rows = _make_streamer(_NU)(eu_t, su_sorted)
    ei_rows = _make_streamer(_NI)(ei_t, si_sorted)

    mesh = plsc.VectorSubcoreMesh(core_axis_name="c", subcore_axis_name="s")
    dot = functools.partial(
        pl.kernel,
        mesh=mesh,
        compiler_params=pltpu.CompilerParams(
            needs_layout_passes=False, use_tc_tiling_on_sc=True),
        out_type=jax.ShapeDtypeStruct((_B,), jnp.float32),
        scratch_types=[
            pltpu.VMEM((_BPW,), jnp.int32),
            pltpu.VMEM((_BPW,), jnp.int32),
            pltpu.VMEM((_BPW,), jnp.int32),
            pltpu.VMEM((_BPW,), jnp.int32),
            pltpu.VMEM((_BPC * _D,), jnp.float32),
            pltpu.VMEM((_BPC * _D,), jnp.float32),
            pltpu.VMEM((_BPW * _L,), jnp.float32),
            pltpu.VMEM((_BPW * _L,), jnp.float32),
            pltpu.VMEM((_BPW,), jnp.float32),
            pltpu.SemaphoreType.DMA,
            pltpu.SemaphoreType.DMA,
        ],
    )(_dot_body)
    return dot(eu_rows, ei_rows, ub_flat, ib_flat, posu, posi,
               user_idx, item_idx)


def kernel(embed_user, embed_item, user_bias, item_bias, user_idx, item_idx):
    return _sc_call(embed_user.T, embed_item.T,
                    user_bias.reshape(-1), item_bias.reshape(-1),
                    user_idx.astype(jnp.int32), item_idx.astype(jnp.int32))
